# baseline (speedup 1.0000x reference)
import jax
import jax.numpy as jnp
from jax.experimental import pallas as pl
from jax.experimental.pallas import tpu as pltpu

_NCHUNK = 100
_NBUF = 8
_W = 4


def _relay(x_hbm, o_hbm, bufs, in_sems, out_sems):
    n = x_hbm.shape[0]
    c = n // _NCHUNK

    def in_cp(i):
        b = i % _NBUF
        return pltpu.make_async_copy(
            x_hbm.at[pl.ds(i * c, c), :], bufs.at[b], in_sems.at[b])

    def out_cp(i):
        b = i % _NBUF
        return pltpu.make_async_copy(
            bufs.at[b], o_hbm.at[pl.ds(i * c, c), :], out_sems.at[b])

    for i in range(_W):
        in_cp(i).start()
    waited_out = 0
    for i in range(_NCHUNK):
        nxt = i + _W
        if nxt < _NCHUNK:
            prev = nxt - _NBUF
            if prev >= 0:
                out_cp(prev).wait()
                waited_out = prev + 1
            in_cp(nxt).start()
        in_cp(i).wait()
        out_cp(i).start()
    for i in range(waited_out, _NCHUNK):
        out_cp(i).wait()


def kernel(x, u):
    n, d = x.shape
    assert n % _NCHUNK == 0 and (n // _NCHUNK) % 8 == 0
    c = n // _NCHUNK
    return pl.pallas_call(
        _relay,
        in_specs=[pl.BlockSpec(memory_space=pl.ANY)],
        out_specs=pl.BlockSpec(memory_space=pl.ANY),
        out_shape=jax.ShapeDtypeStruct((n, d), x.dtype),
        scratch_shapes=[
            pltpu.VMEM((_NBUF, c, d), jnp.float32),
            pltpu.SemaphoreType.DMA((_NBUF,)),
            pltpu.SemaphoreType.DMA((_NBUF,)),
        ],
    )(x)


# speedup vs baseline: 1.0023x; 1.0023x over previous
import jax
import jax.numpy as jnp
from jax.experimental import pallas as pl
from jax.experimental.pallas import tpu as pltpu

_NCHUNK = 25
_NBUF = 8
_W = 4


def _relay(x_hbm, o_hbm, bufs, in_sems, out_sems):
    n = x_hbm.shape[0]
    c = n // _NCHUNK

    def in_cp(i):
        b = i % _NBUF
        return pltpu.make_async_copy(
            x_hbm.at[pl.ds(i * c, c), :], bufs.at[b], in_sems.at[b])

    def out_cp(i):
        b = i % _NBUF
        return pltpu.make_async_copy(
            bufs.at[b], o_hbm.at[pl.ds(i * c, c), :], out_sems.at[b])

    for i in range(_W):
        in_cp(i).start()
    waited_out = 0
    for i in range(_NCHUNK):
        nxt = i + _W
        if nxt < _NCHUNK:
            prev = nxt - _NBUF
            if prev >= 0:
                out_cp(prev).wait()
                waited_out = prev + 1
            in_cp(nxt).start()
        in_cp(i).wait()
        out_cp(i).start()
    for i in range(waited_out, _NCHUNK):
        out_cp(i).wait()


def kernel(x, u):
    n, d = x.shape
    assert n % _NCHUNK == 0 and (n // _NCHUNK) % 8 == 0
    c = n // _NCHUNK
    return pl.pallas_call(
        _relay,
        in_specs=[pl.BlockSpec(memory_space=pl.ANY)],
        out_specs=pl.BlockSpec(memory_space=pl.ANY),
        out_shape=jax.ShapeDtypeStruct((n, d), x.dtype),
        scratch_shapes=[
            pltpu.VMEM((_NBUF, c, d), jnp.float32),
            pltpu.SemaphoreType.DMA((_NBUF,)),
            pltpu.SemaphoreType.DMA((_NBUF,)),
        ],
    )(x)
